# Initial kernel scaffold; baseline (speedup 1.0000x reference)
import jax
import jax.numpy as jnp
from jax.experimental import pallas as pl


def kernel(x, task_ids, params):
    raise NotImplementedError("write your pallas kernel here")


# speedup vs baseline: 2.4959x; 2.4959x over previous
import functools
import math

import jax
import jax.numpy as jnp
import numpy as np
from jax.experimental import pallas as pl

D = 128
DFF = 512
E = 8
K = 2
H = 4
T = 5
NSEG = 30
SEGLEN = 250
NFFT = 64
HOP = 32
NFREQ = 33
NFRAMES = 8
FLAT = NFREQ * NFRAMES
NTOK = 8 * NSEG
NPAD = 256
NREAL = NTOK + 1


def _make_dft_mats():
    x = np.arange(NFFT)[:, None]
    q = np.arange(NFREQ)[None, :]
    ang = 2.0 * np.pi * x * q / NFFT
    C = np.cos(ang).astype(np.float32)
    S = np.sin(ang).astype(np.float32)
    Cb = np.zeros((NFRAMES * NFFT, NFRAMES * NFREQ), np.float32)
    Sb = np.zeros((NFRAMES * NFFT, NFRAMES * NFREQ), np.float32)
    for fr in range(NFRAMES):
        Cb[fr * NFFT:(fr + 1) * NFFT, fr * NFREQ:(fr + 1) * NFREQ] = C
        Sb[fr * NFFT:(fr + 1) * NFFT, fr * NFREQ:(fr + 1) * NFREQ] = S
    return Cb, Sb


_CB, _SB = _make_dft_mats()

_PERM = np.array([q * NFRAMES + fr for fr in range(NFRAMES) for q in range(NFREQ)],
                 dtype=np.int32)


def _gelu(x):
    return 0.5 * x * (1.0 + jax.lax.erf(x * (1.0 / math.sqrt(2.0))))


def _ln(x, g, b, eps=1e-5):
    m = jnp.mean(x, axis=-1, keepdims=True)
    v = jnp.mean((x - m) ** 2, axis=-1, keepdims=True)
    return (x - m) * jax.lax.rsqrt(v + eps) * g + b


def _tok_kernel(xp_ref, cb_ref, sb_ref, w2_ref, bp_ref, g_ref, b_ref, out_ref):
    xp = xp_ref[...]
    frames = jnp.concatenate(
        [xp[:, fr * HOP: fr * HOP + NFFT] for fr in range(NFRAMES)], axis=1)
    re = jnp.dot(frames, cb_ref[...], preferred_element_type=jnp.float32)
    im = jnp.dot(frames, sb_ref[...], preferred_element_type=jnp.float32)
    mag = jnp.sqrt(re * re + im * im)
    tok = jnp.dot(mag, w2_ref[...], preferred_element_type=jnp.float32) + bp_ref[...]
    tok = _ln(tok, g_ref[...], b_ref[...])
    out_ref[...] = _gelu(tok)


def _tx_kernel(src_ref, oh_ref,
               n1g_ref, n1b_ref, n2g_ref, n2b_ref, nfg_ref, nfb_ref,
               wq_ref, bq_ref, wk_ref, bk_ref, wv_ref, bv_ref, wo_ref, bo_ref,
               te_ref, wg1_ref, wg2_ref, bg_ref,
               we1_ref, be1_ref, we2_ref, be2_ref,
               wu1_ref, bu1_ref, wu2_ref, bu2_ref,
               hw_ref, hb_ref,
               gl_ref, tl_ref):
    src = src_ref[0]
    dh = D // H

    s2 = _ln(src, n1g_ref[...], n1b_ref[...])
    q = jnp.dot(s2, wq_ref[...], preferred_element_type=jnp.float32) + bq_ref[...]
    k = jnp.dot(s2, wk_ref[...], preferred_element_type=jnp.float32) + bk_ref[...]
    v = jnp.dot(s2, wv_ref[...], preferred_element_type=jnp.float32) + bv_ref[...]

    col = jax.lax.broadcasted_iota(jnp.int32, (NPAD, NPAD), 1)
    key_mask = col < NREAL
    scale = 1.0 / math.sqrt(dh)
    heads = []
    for h in range(H):
        sl = slice(h * dh, (h + 1) * dh)
        sc = jnp.dot(q[:, sl], k[:, sl].T, preferred_element_type=jnp.float32) * scale
        sc = jnp.where(key_mask, sc, -1e30)
        m = jnp.max(sc, axis=-1, keepdims=True)
        p = jnp.exp(sc - m)
        p = p / jnp.sum(p, axis=-1, keepdims=True)
        heads.append(jnp.dot(p, v[:, sl], preferred_element_type=jnp.float32))
    ao = jnp.dot(jnp.concatenate(heads, axis=1), wo_ref[...],
                 preferred_element_type=jnp.float32) + bo_ref[...]
    src = src + ao

    s2 = _ln(src, n2g_ref[...], n2b_ref[...])
    oh = oh_ref[0]
    tvec = jnp.dot(oh, te_ref[...], preferred_element_type=jnp.float32)
    gl = (jnp.dot(s2, wg1_ref[...], preferred_element_type=jnp.float32)
          + jnp.dot(tvec, wg2_ref[...], preferred_element_type=jnp.float32)
          + bg_ref[...])
    gl_ref[0] = gl

    eidx = jax.lax.broadcasted_iota(jnp.int32, (NPAD, E), 1)
    m1 = jnp.max(gl, axis=-1, keepdims=True)
    i1 = jnp.min(jnp.where(gl == m1, eidx, E), axis=-1, keepdims=True)
    oh1 = eidx == i1
    gl2 = jnp.where(oh1, -1e30, gl)
    m2 = jnp.max(gl2, axis=-1, keepdims=True)
    i2 = jnp.min(jnp.where(gl2 == m2, eidx, E), axis=-1, keepdims=True)
    sel = oh1 | (eidx == i2)
    es = jnp.where(sel, jnp.exp(gl - m1), 0.0)
    z = jnp.sum(es, axis=-1, keepdims=True)
    gates = es / z
    omega = 1.0 - jnp.max(gates, axis=-1, keepdims=True)

    acc = jnp.zeros((NPAD, D), jnp.float32)
    for e in range(E):
        he = _gelu(jnp.dot(s2, we1_ref[e], preferred_element_type=jnp.float32)
                   + be1_ref[e])
        eo = jnp.dot(he, we2_ref[e], preferred_element_type=jnp.float32) + be2_ref[e]
        acc = acc + gates[:, e:e + 1] * eo

    univ = jnp.dot(_gelu(jnp.dot(s2, wu1_ref[...], preferred_element_type=jnp.float32)
                         + bu1_ref[...]),
                   wu2_ref[...], preferred_element_type=jnp.float32) + bu2_ref[...]

    src = src + acc + omega * univ
    out = _ln(src, nfg_ref[...], nfb_ref[...])
    cls = out[0:1, :]

    a = jnp.concatenate(
        [jnp.dot(cls, hw_ref[t], preferred_element_type=jnp.float32)
         for t in range(T)], axis=0)
    tl = (jnp.dot(oh[:, :T], a, preferred_element_type=jnp.float32)
          + jnp.dot(oh[:, :T], hb_ref[...], preferred_element_type=jnp.float32))
    tl_ref[0] = tl


def kernel(x, task_ids, params):
    p = params
    B, C, L = x.shape
    f32 = jnp.float32

    xs = x.reshape(B * C * NSEG, SEGLEN)
    xp = jnp.pad(xs, ((0, 0), (NFFT // 2, NFFT // 2)), mode='reflect')
    w2 = p['W_proj'][_PERM]
    row2 = lambda a: a.reshape(1, -1)

    TB = 512
    ntok_total = B * C * NSEG
    grid1 = ntok_total // TB
    tokens = pl.pallas_call(
        _tok_kernel,
        grid=(grid1,),
        in_specs=[
            pl.BlockSpec((TB, SEGLEN + NFFT), lambda i: (i, 0)),
            pl.BlockSpec((NFRAMES * NFFT, FLAT), lambda i: (0, 0)),
            pl.BlockSpec((NFRAMES * NFFT, FLAT), lambda i: (0, 0)),
            pl.BlockSpec((FLAT, D), lambda i: (0, 0)),
            pl.BlockSpec((1, D), lambda i: (0, 0)),
            pl.BlockSpec((1, D), lambda i: (0, 0)),
            pl.BlockSpec((1, D), lambda i: (0, 0)),
        ],
        out_specs=pl.BlockSpec((TB, D), lambda i: (i, 0)),
        out_shape=jax.ShapeDtypeStruct((ntok_total, D), f32),
    )(xp, jnp.asarray(_CB), jnp.asarray(_SB), w2,
      row2(p['b_proj']), row2(p['ln_proj_g']), row2(p['ln_proj_b']))

    tokens = tokens.reshape(B, NTOK, D) + p['pos_embed']
    cls = jnp.broadcast_to(p['cls_token'], (B, 1, D))
    src = jnp.concatenate([cls, tokens], axis=1)
    src = jnp.pad(src, ((0, 0), (0, NPAD - NREAL), (0, 0)))

    oh = jax.nn.one_hot(task_ids, E, dtype=f32).reshape(B, 1, E)
    te_pad = jnp.pad(p['task_embed'], ((0, E - T), (0, 0)))
    wg1 = p['Wg'][:D]
    wg2 = p['Wg'][D:]

    const = lambda shape: pl.BlockSpec(shape, lambda b: tuple(0 for _ in shape))
    gl_out, tl_out = pl.pallas_call(
        _tx_kernel,
        grid=(B,),
        in_specs=[
            pl.BlockSpec((1, NPAD, D), lambda b: (b, 0, 0)),
            pl.BlockSpec((1, 1, E), lambda b: (b, 0, 0)),
            const((1, D)), const((1, D)), const((1, D)), const((1, D)),
            const((1, D)), const((1, D)),
            const((D, D)), const((1, D)), const((D, D)), const((1, D)),
            const((D, D)), const((1, D)), const((D, D)), const((1, D)),
            const((E, D)), const((D, E)), const((D, E)), const((1, E)),
            const((E, D, DFF)), const((E, 1, DFF)),
            const((E, DFF, D)), const((E, 1, D)),
            const((D, DFF)), const((1, DFF)), const((DFF, D)), const((1, D)),
            const((T, D, 2)), const((T, 2)),
        ],
        out_specs=[
            pl.BlockSpec((1, NPAD, E), lambda b: (b, 0, 0)),
            pl.BlockSpec((1, 1, 2), lambda b: (b, 0, 0)),
        ],
        out_shape=[
            jax.ShapeDtypeStruct((B, NPAD, E), f32),
            jax.ShapeDtypeStruct((B, 1, 2), f32),
        ],
    )(src, oh,
      row2(p['norm1_g']), row2(p['norm1_b']),
      row2(p['norm2_g']), row2(p['norm2_b']),
      row2(p['normf_g']), row2(p['normf_b']),
      p['Wq'], row2(p['bq']), p['Wk'], row2(p['bk']),
      p['Wv'], row2(p['bv']), p['Wo'], row2(p['bo']),
      te_pad, wg1, wg2, row2(p['bg']),
      p['We1'], p['be1'].reshape(E, 1, DFF),
      p['We2'], p['be2'].reshape(E, 1, D),
      p['Wu1'], row2(p['bu1']), p['Wu2'], row2(p['bu2']),
      p['head_W'], p['head_b'])

    task_logits = tl_out.reshape(B, 2)
    logits = gl_out[:, :NREAL, :]
    return task_logits, logits


# speedup vs baseline: 2.5653x; 1.0278x over previous
import math

import jax
import jax.numpy as jnp
import numpy as np
from jax.experimental import pallas as pl

D = 128
DFF = 512
E = 8
H = 4
T = 5
NSEG = 30
SEGLEN = 250
NFFT = 64
HOP = 32
NFREQ = 33
NFRAMES = 8
FLAT = NFREQ * NFRAMES
NTOK = 8 * NSEG
NPAD = 256
NREAL = NTOK + 1
PADLEN = SEGLEN + NFFT


def _make_dft_mats():
    x = np.arange(NFFT)[:, None]
    q = np.arange(NFREQ)[None, :]
    ang = 2.0 * np.pi * x * q / NFFT
    C = np.cos(ang).astype(np.float32)
    S = np.sin(ang).astype(np.float32)
    Cb = np.zeros((NFRAMES * NFFT, NFRAMES * NFREQ), np.float32)
    Sb = np.zeros((NFRAMES * NFFT, NFRAMES * NFREQ), np.float32)
    for fr in range(NFRAMES):
        Cb[fr * NFFT:(fr + 1) * NFFT, fr * NFREQ:(fr + 1) * NFREQ] = C
        Sb[fr * NFFT:(fr + 1) * NFFT, fr * NFREQ:(fr + 1) * NFREQ] = S
    return Cb, Sb


_CB, _SB = _make_dft_mats()

_PERM = np.array([q * NFRAMES + fr for fr in range(NFRAMES) for q in range(NFREQ)],
                 dtype=np.int32)


def _gelu(x):
    return 0.5 * x * (1.0 + jax.lax.erf(x * (1.0 / math.sqrt(2.0))))


def _ln(x, g, b, eps=1e-5):
    m = jnp.mean(x, axis=-1, keepdims=True)
    v = jnp.mean((x - m) ** 2, axis=-1, keepdims=True)
    return (x - m) * jax.lax.rsqrt(v + eps) * g + b


def _fused_kernel(xp_ref, oh_ref,
                  cb_ref, sb_ref, w2_ref, bp_ref, lng_ref, lnb_ref,
                  pos_ref, cls_ref,
                  n1g_ref, n1b_ref, n2g_ref, n2b_ref, nfg_ref, nfb_ref,
                  wq_ref, bq_ref, wk_ref, bk_ref, wv_ref, bv_ref,
                  wo_ref, bo_ref,
                  te_ref, wg1_ref, wg2_ref, bg_ref,
                  we1_ref, be1_ref, we2_ref, be2_ref,
                  wu1_ref, bu1_ref, wu2_ref, bu2_ref,
                  hw_ref, hb_ref,
                  gl_ref, tl_ref):
    dh = D // H
    f32 = jnp.float32

    xp = xp_ref[0]
    frames = jnp.concatenate(
        [xp[:, fr * HOP: fr * HOP + NFFT] for fr in range(NFRAMES)], axis=1)
    re = jnp.dot(frames, cb_ref[...], preferred_element_type=f32)
    im = jnp.dot(frames, sb_ref[...], preferred_element_type=f32)
    mag = jnp.sqrt(re * re + im * im)
    tok = jnp.dot(mag, w2_ref[...], preferred_element_type=f32) + bp_ref[...]
    tok = _gelu(_ln(tok, lng_ref[...], lnb_ref[...])) + pos_ref[0]

    src = jnp.concatenate(
        [cls_ref[...], tok, jnp.zeros((NPAD - NREAL, D), f32)], axis=0)

    s2 = _ln(src, n1g_ref[...], n1b_ref[...])
    q = jnp.dot(s2, wq_ref[...], preferred_element_type=f32) + bq_ref[...]
    k = jnp.dot(s2, wk_ref[...], preferred_element_type=f32) + bk_ref[...]
    v = jnp.dot(s2, wv_ref[...], preferred_element_type=f32) + bv_ref[...]

    col = jax.lax.broadcasted_iota(jnp.int32, (NPAD, NPAD), 1)
    key_mask = col < NREAL
    scale = 1.0 / math.sqrt(dh)
    heads = []
    for h in range(H):
        sl = slice(h * dh, (h + 1) * dh)
        sc = jnp.dot(q[:, sl], k[:, sl].T, preferred_element_type=f32) * scale
        sc = jnp.where(key_mask, sc, -1e30)
        m = jnp.max(sc, axis=-1, keepdims=True)
        p = jnp.exp(sc - m)
        p = p / jnp.sum(p, axis=-1, keepdims=True)
        heads.append(jnp.dot(p, v[:, sl], preferred_element_type=f32))
    ao = jnp.dot(jnp.concatenate(heads, axis=1), wo_ref[...],
                 preferred_element_type=f32) + bo_ref[...]
    src = src + ao

    s2 = _ln(src, n2g_ref[...], n2b_ref[...])
    oh = oh_ref[0]
    tvec = jnp.dot(oh, te_ref[...], preferred_element_type=f32)
    gl = (jnp.dot(s2, wg1_ref[...], preferred_element_type=f32)
          + jnp.dot(tvec, wg2_ref[...], preferred_element_type=f32)
          + bg_ref[...])
    gl_ref[0] = gl

    eidx = jax.lax.broadcasted_iota(jnp.int32, (NPAD, E), 1)
    m1 = jnp.max(gl, axis=-1, keepdims=True)
    i1 = jnp.min(jnp.where(gl == m1, eidx, E), axis=-1, keepdims=True)
    oh1 = eidx == i1
    gl2 = jnp.where(oh1, -1e30, gl)
    m2 = jnp.max(gl2, axis=-1, keepdims=True)
    i2 = jnp.min(jnp.where(gl2 == m2, eidx, E), axis=-1, keepdims=True)
    sel = oh1 | (eidx == i2)
    es = jnp.where(sel, jnp.exp(gl - m1), 0.0)
    z = jnp.sum(es, axis=-1, keepdims=True)
    gates = es / z
    omega = 1.0 - jnp.max(gates, axis=-1, keepdims=True)

    acc = jnp.zeros((NPAD, D), f32)
    for e in range(E):
        he = _gelu(jnp.dot(s2, we1_ref[e], preferred_element_type=f32)
                   + be1_ref[e])
        eo = jnp.dot(he, we2_ref[e], preferred_element_type=f32) + be2_ref[e]
        acc = acc + gates[:, e:e + 1] * eo

    univ = jnp.dot(_gelu(jnp.dot(s2, wu1_ref[...], preferred_element_type=f32)
                         + bu1_ref[...]),
                   wu2_ref[...], preferred_element_type=f32) + bu2_ref[...]

    src = src + acc + omega * univ
    out = _ln(src, nfg_ref[...], nfb_ref[...])
    cls = out[0:1, :]

    a = jnp.concatenate(
        [jnp.dot(cls, hw_ref[t], preferred_element_type=f32)
         for t in range(T)], axis=0)
    tl = (jnp.dot(oh[:, :T], a, preferred_element_type=f32)
          + jnp.dot(oh[:, :T], hb_ref[...], preferred_element_type=f32))
    tl_ref[0] = tl


def kernel(x, task_ids, params):
    p = params
    B = x.shape[0]
    f32 = jnp.float32

    xs = x.reshape(B, NTOK, SEGLEN)
    xp = jnp.pad(xs, ((0, 0), (0, 0), (NFFT // 2, NFFT // 2)), mode='reflect')

    w2 = p['W_proj'][_PERM]
    row2 = lambda a: a.reshape(1, -1)
    oh = jax.nn.one_hot(task_ids, E, dtype=f32).reshape(B, 1, E)
    te_pad = jnp.pad(p['task_embed'], ((0, E - T), (0, 0)))
    wg1 = p['Wg'][:D]
    wg2 = p['Wg'][D:]

    const = lambda shape: pl.BlockSpec(shape, lambda b: tuple(0 for _ in shape))
    gl_out, tl_out = pl.pallas_call(
        _fused_kernel,
        grid=(B,),
        in_specs=[
            pl.BlockSpec((1, NTOK, PADLEN), lambda b: (b, 0, 0)),
            pl.BlockSpec((1, 1, E), lambda b: (b, 0, 0)),
            const((NFRAMES * NFFT, FLAT)), const((NFRAMES * NFFT, FLAT)),
            const((FLAT, D)), const((1, D)), const((1, D)), const((1, D)),
            const((1, NTOK, D)), const((1, D)),
            const((1, D)), const((1, D)), const((1, D)), const((1, D)),
            const((1, D)), const((1, D)),
            const((D, D)), const((1, D)), const((D, D)), const((1, D)),
            const((D, D)), const((1, D)), const((D, D)), const((1, D)),
            const((E, D)), const((D, E)), const((D, E)), const((1, E)),
            const((E, D, DFF)), const((E, 1, DFF)),
            const((E, DFF, D)), const((E, 1, D)),
            const((D, DFF)), const((1, DFF)), const((DFF, D)), const((1, D)),
            const((T, D, 2)), const((T, 2)),
        ],
        out_specs=[
            pl.BlockSpec((1, NPAD, E), lambda b: (b, 0, 0)),
            pl.BlockSpec((1, 1, 2), lambda b: (b, 0, 0)),
        ],
        out_shape=[
            jax.ShapeDtypeStruct((B, NPAD, E), f32),
            jax.ShapeDtypeStruct((B, 1, 2), f32),
        ],
    )(xp, oh,
      jnp.asarray(_CB), jnp.asarray(_SB), w2,
      row2(p['b_proj']), row2(p['ln_proj_g']), row2(p['ln_proj_b']),
      p['pos_embed'], p['cls_token'].reshape(1, D),
      row2(p['norm1_g']), row2(p['norm1_b']),
      row2(p['norm2_g']), row2(p['norm2_b']),
      row2(p['normf_g']), row2(p['normf_b']),
      p['Wq'], row2(p['bq']), p['Wk'], row2(p['bk']),
      p['Wv'], row2(p['bv']), p['Wo'], row2(p['bo']),
      te_pad, wg1, wg2, row2(p['bg']),
      p['We1'], p['be1'].reshape(E, 1, DFF),
      p['We2'], p['be2'].reshape(E, 1, D),
      p['Wu1'], row2(p['bu1']), p['Wu2'], row2(p['bu2']),
      p['head_W'], p['head_b'])

    task_logits = tl_out.reshape(B, 2)
    logits = gl_out[:, :NREAL, :]
    return task_logits, logits


# speedup vs baseline: 2.8766x; 1.1214x over previous
import math

import jax
import jax.numpy as jnp
import numpy as np
from jax.experimental import pallas as pl

D = 128
DFF = 512
E = 8
H = 4
T = 5
NSEG = 30
SEGLEN = 250
NFFT = 64
HOP = 32
NFREQ = 33
NFRAMES = 8
FLAT = NFREQ * NFRAMES
NTOK = 8 * NSEG
NPAD = 256
NREAL = NTOK + 1
PADLEN = SEGLEN + NFFT


def _make_dft_mats():
    x = np.arange(NFFT)[:, None]
    q = np.arange(NFREQ)[None, :]
    ang = 2.0 * np.pi * x * q / NFFT
    C = np.cos(ang).astype(np.float32)
    S = np.sin(ang).astype(np.float32)
    Cb = np.zeros((NFRAMES * NFFT, NFRAMES * NFREQ), np.float32)
    Sb = np.zeros((NFRAMES * NFFT, NFRAMES * NFREQ), np.float32)
    for fr in range(NFRAMES):
        Cb[fr * NFFT:(fr + 1) * NFFT, fr * NFREQ:(fr + 1) * NFREQ] = C
        Sb[fr * NFFT:(fr + 1) * NFFT, fr * NFREQ:(fr + 1) * NFREQ] = S
    return Cb, Sb


_CB, _SB = _make_dft_mats()

_PERM = np.array([q * NFRAMES + fr for fr in range(NFRAMES) for q in range(NFREQ)],
                 dtype=np.int32)


def _gelu(x):
    return 0.5 * x * (1.0 + jax.lax.erf(x * (1.0 / math.sqrt(2.0))))


def _ln(x, g, b, eps=1e-5):
    m = jnp.mean(x, axis=-1, keepdims=True)
    v = jnp.mean((x - m) ** 2, axis=-1, keepdims=True)
    return (x - m) * jax.lax.rsqrt(v + eps) * g + b


def _fused_kernel(xs_ref, lpad_ref, rpad_ref, oh_ref,
                  cb_ref, sb_ref, w2_ref, bp_ref, lng_ref, lnb_ref,
                  pos_ref, cls_ref,
                  n1g_ref, n1b_ref, n2g_ref, n2b_ref, nfg_ref, nfb_ref,
                  wq_ref, bq_ref, wk_ref, bk_ref, wv_ref, bv_ref,
                  wo_ref, bo_ref,
                  te_ref, wg1_ref, wg2_ref, bg_ref,
                  we1_ref, be1_ref, we2_ref, be2_ref,
                  wu1_ref, bu1_ref, wu2_ref, bu2_ref,
                  hw_ref, hb_ref,
                  gl_ref, tl_ref):
    dh = D // H
    f32 = jnp.float32

    xs = xs_ref[0]
    parts = [lpad_ref[0], xs[:, 0:HOP]]
    for fr in range(1, NFRAMES - 1):
        parts.append(xs[:, fr * HOP - HOP: fr * HOP + HOP])
    parts.append(xs[:, (NFRAMES - 1) * HOP - HOP: SEGLEN])
    parts.append(rpad_ref[0])
    frames = jnp.concatenate(parts, axis=1)
    re = jnp.dot(frames, cb_ref[...], preferred_element_type=f32)
    im = jnp.dot(frames, sb_ref[...], preferred_element_type=f32)
    mag = jnp.sqrt(re * re + im * im)
    tok = jnp.dot(mag, w2_ref[...], preferred_element_type=f32) + bp_ref[...]
    tok = _gelu(_ln(tok, lng_ref[...], lnb_ref[...])) + pos_ref[0]

    src = jnp.concatenate(
        [cls_ref[...], tok, jnp.zeros((NPAD - NREAL, D), f32)], axis=0)

    s2 = _ln(src, n1g_ref[...], n1b_ref[...])
    q = jnp.dot(s2, wq_ref[...], preferred_element_type=f32) + bq_ref[...]
    k = jnp.dot(s2, wk_ref[...], preferred_element_type=f32) + bk_ref[...]
    v = jnp.dot(s2, wv_ref[...], preferred_element_type=f32) + bv_ref[...]

    col = jax.lax.broadcasted_iota(jnp.int32, (NPAD, NPAD), 1)
    key_mask = col < NREAL
    scale = 1.0 / math.sqrt(dh)
    heads = []
    for h in range(H):
        sl = slice(h * dh, (h + 1) * dh)
        sc = jnp.dot(q[:, sl], k[:, sl].T, preferred_element_type=f32) * scale
        sc = jnp.where(key_mask, sc, -1e30)
        m = jnp.max(sc, axis=-1, keepdims=True)
        p = jnp.exp(sc - m)
        p = p / jnp.sum(p, axis=-1, keepdims=True)
        heads.append(jnp.dot(p, v[:, sl], preferred_element_type=f32))
    ao = jnp.dot(jnp.concatenate(heads, axis=1), wo_ref[...],
                 preferred_element_type=f32) + bo_ref[...]
    src = src + ao

    s2 = _ln(src, n2g_ref[...], n2b_ref[...])
    oh = oh_ref[0]
    tvec = jnp.dot(oh, te_ref[...], preferred_element_type=f32)
    gl = (jnp.dot(s2, wg1_ref[...], preferred_element_type=f32)
          + jnp.dot(tvec, wg2_ref[...], preferred_element_type=f32)
          + bg_ref[...])
    gl_ref[0] = gl

    eidx = jax.lax.broadcasted_iota(jnp.int32, (NPAD, E), 1)
    m1 = jnp.max(gl, axis=-1, keepdims=True)
    i1 = jnp.min(jnp.where(gl == m1, eidx, E), axis=-1, keepdims=True)
    oh1 = eidx == i1
    gl2 = jnp.where(oh1, -1e30, gl)
    m2 = jnp.max(gl2, axis=-1, keepdims=True)
    i2 = jnp.min(jnp.where(gl2 == m2, eidx, E), axis=-1, keepdims=True)
    sel = oh1 | (eidx == i2)
    es = jnp.where(sel, jnp.exp(gl - m1), 0.0)
    z = jnp.sum(es, axis=-1, keepdims=True)
    gates = es / z
    omega = 1.0 - jnp.max(gates, axis=-1, keepdims=True)

    acc = jnp.zeros((NPAD, D), f32)
    for e in range(E):
        he = _gelu(jnp.dot(s2, we1_ref[e], preferred_element_type=f32)
                   + be1_ref[e])
        eo = jnp.dot(he, we2_ref[e], preferred_element_type=f32) + be2_ref[e]
        acc = acc + gates[:, e:e + 1] * eo

    univ = jnp.dot(_gelu(jnp.dot(s2, wu1_ref[...], preferred_element_type=f32)
                         + bu1_ref[...]),
                   wu2_ref[...], preferred_element_type=f32) + bu2_ref[...]

    src = src + acc + omega * univ
    out = _ln(src, nfg_ref[...], nfb_ref[...])
    cls = out[0:1, :]

    a = jnp.concatenate(
        [jnp.dot(cls, hw_ref[t], preferred_element_type=f32)
         for t in range(T)], axis=0)
    tl = (jnp.dot(oh[:, :T], a, preferred_element_type=f32)
          + jnp.dot(oh[:, :T], hb_ref[...], preferred_element_type=f32))
    tl_ref[0] = tl


def kernel(x, task_ids, params):
    p = params
    B = x.shape[0]
    f32 = jnp.float32

    xs = x.reshape(B, NTOK, SEGLEN)
    lpad = xs[:, :, NFFT // 2:0:-1]
    rpad = xs[:, :, SEGLEN - 2:SEGLEN - 2 - (NFFT // 2 - (SEGLEN - HOP * (NFRAMES - 1))):-1]

    w2 = p['W_proj'][_PERM]
    row2 = lambda a: a.reshape(1, -1)
    oh = jax.nn.one_hot(task_ids, E, dtype=f32).reshape(B, 1, E)
    te_pad = jnp.pad(p['task_embed'], ((0, E - T), (0, 0)))
    wg1 = p['Wg'][:D]
    wg2 = p['Wg'][D:]

    const = lambda shape: pl.BlockSpec(shape, lambda b: tuple(0 for _ in shape))
    gl_out, tl_out = pl.pallas_call(
        _fused_kernel,
        grid=(B,),
        in_specs=[
            pl.BlockSpec((1, NTOK, SEGLEN), lambda b: (b, 0, 0)),
            pl.BlockSpec((1, NTOK, HOP), lambda b: (b, 0, 0)),
            pl.BlockSpec((1, NTOK, 6), lambda b: (b, 0, 0)),
            pl.BlockSpec((1, 1, E), lambda b: (b, 0, 0)),
            const((NFRAMES * NFFT, FLAT)), const((NFRAMES * NFFT, FLAT)),
            const((FLAT, D)), const((1, D)), const((1, D)), const((1, D)),
            const((1, NTOK, D)), const((1, D)),
            const((1, D)), const((1, D)), const((1, D)), const((1, D)),
            const((1, D)), const((1, D)),
            const((D, D)), const((1, D)), const((D, D)), const((1, D)),
            const((D, D)), const((1, D)), const((D, D)), const((1, D)),
            const((E, D)), const((D, E)), const((D, E)), const((1, E)),
            const((E, D, DFF)), const((E, 1, DFF)),
            const((E, DFF, D)), const((E, 1, D)),
            const((D, DFF)), const((1, DFF)), const((DFF, D)), const((1, D)),
            const((T, D, 2)), const((T, 2)),
        ],
        out_specs=[
            pl.BlockSpec((1, NPAD, E), lambda b: (b, 0, 0)),
            pl.BlockSpec((1, 1, 2), lambda b: (b, 0, 0)),
        ],
        out_shape=[
            jax.ShapeDtypeStruct((B, NPAD, E), f32),
            jax.ShapeDtypeStruct((B, 1, 2), f32),
        ],
    )(xs, lpad, rpad, oh,
      jnp.asarray(_CB), jnp.asarray(_SB), w2,
      row2(p['b_proj']), row2(p['ln_proj_g']), row2(p['ln_proj_b']),
      p['pos_embed'], p['cls_token'].reshape(1, D),
      row2(p['norm1_g']), row2(p['norm1_b']),
      row2(p['norm2_g']), row2(p['norm2_b']),
      row2(p['normf_g']), row2(p['normf_b']),
      p['Wq'], row2(p['bq']), p['Wk'], row2(p['bk']),
      p['Wv'], row2(p['bv']), p['Wo'], row2(p['bo']),
      te_pad, wg1, wg2, row2(p['bg']),
      p['We1'], p['be1'].reshape(E, 1, DFF),
      p['We2'], p['be2'].reshape(E, 1, D),
      p['Wu1'], row2(p['bu1']), p['Wu2'], row2(p['bu2']),
      p['head_W'], p['head_b'])

    task_logits = tl_out.reshape(B, 2)
    logits = gl_out[:, :NREAL, :]
    return task_logits, logits


# speedup vs baseline: 3.9003x; 1.3558x over previous
import math

import jax
import jax.numpy as jnp
import numpy as np
from jax.experimental import pallas as pl

D = 128
DFF = 512
E = 8
H = 4
T = 5
NSEG = 30
SEGLEN = 250
NFFT = 64
HOP = 32
NFREQ = 33
NFRAMES = 8
FLAT = NFREQ * NFRAMES
NTOK = 8 * NSEG
NPAD = 256
NREAL = NTOK + 1
PADLEN = SEGLEN + NFFT


def _make_dft_mats():
    x = np.arange(NFFT)
    q = np.arange(NFREQ)[None, :]
    ang = 2.0 * np.pi * x[:, None] * q / NFFT
    C = np.cos(ang)
    S = np.sin(ang)
    CF = np.zeros((SEGLEN, FLAT), np.float64)
    SF = np.zeros((SEGLEN, FLAT), np.float64)
    for fr in range(NFRAMES):
        for xi in range(NFFT):
            jp = fr * HOP + xi
            if jp < NFFT // 2:
                si = NFFT // 2 - jp
            elif jp < NFFT // 2 + SEGLEN:
                si = jp - NFFT // 2
            else:
                si = (SEGLEN - 2) - (jp - (NFFT // 2 + SEGLEN))
            CF[si, q[0] * NFRAMES + fr] += C[xi]
            SF[si, q[0] * NFRAMES + fr] += S[xi]
    return CF.astype(np.float32), SF.astype(np.float32)


_CF, _SF = _make_dft_mats()


def _gelu(x):
    return 0.5 * x * (1.0 + jax.lax.erf(x * (1.0 / math.sqrt(2.0))))


def _ln(x, g, b, eps=1e-5):
    m = jnp.mean(x, axis=-1, keepdims=True)
    v = jnp.mean((x - m) ** 2, axis=-1, keepdims=True)
    return (x - m) * jax.lax.rsqrt(v + eps) * g + b


def _fused_kernel(xs_ref, oh_ref,
                  cb_ref, sb_ref, w2_ref, bp_ref, lng_ref, lnb_ref,
                  pos_ref, cls_ref,
                  n1g_ref, n1b_ref, n2g_ref, n2b_ref, nfg_ref, nfb_ref,
                  wq_ref, bq_ref, wk_ref, bk_ref, wv_ref, bv_ref,
                  wo_ref, bo_ref,
                  te_ref, wg1_ref, wg2_ref, bg_ref,
                  we1_ref, be1_ref, we2_ref, be2_ref,
                  wu1_ref, bu1_ref, wu2_ref, bu2_ref,
                  hw_ref, hb_ref,
                  gl_ref, tl_ref):
    dh = D // H
    f32 = jnp.float32

    xs = xs_ref[0]
    re = jnp.dot(xs, cb_ref[...], preferred_element_type=f32)
    im = jnp.dot(xs, sb_ref[...], preferred_element_type=f32)
    mag = jnp.sqrt(re * re + im * im)
    tok = jnp.dot(mag, w2_ref[...], preferred_element_type=f32) + bp_ref[...]
    tok = _gelu(_ln(tok, lng_ref[...], lnb_ref[...])) + pos_ref[0]

    src = jnp.concatenate(
        [cls_ref[...], tok, jnp.zeros((NPAD - NREAL, D), f32)], axis=0)

    s2 = _ln(src, n1g_ref[...], n1b_ref[...])
    q = jnp.dot(s2, wq_ref[...], preferred_element_type=f32) + bq_ref[...]
    k = jnp.dot(s2, wk_ref[...], preferred_element_type=f32) + bk_ref[...]
    v = jnp.dot(s2, wv_ref[...], preferred_element_type=f32) + bv_ref[...]

    col = jax.lax.broadcasted_iota(jnp.int32, (NPAD, NPAD), 1)
    key_mask = col < NREAL
    scale = 1.0 / math.sqrt(dh)
    heads = []
    for h in range(H):
        sl = slice(h * dh, (h + 1) * dh)
        sc = jnp.dot(q[:, sl], k[:, sl].T, preferred_element_type=f32) * scale
        sc = jnp.where(key_mask, sc, -1e30)
        m = jnp.max(sc, axis=-1, keepdims=True)
        p = jnp.exp(sc - m)
        p = p / jnp.sum(p, axis=-1, keepdims=True)
        heads.append(jnp.dot(p, v[:, sl], preferred_element_type=f32))
    ao = jnp.dot(jnp.concatenate(heads, axis=1), wo_ref[...],
                 preferred_element_type=f32) + bo_ref[...]
    src = src + ao

    s2 = _ln(src, n2g_ref[...], n2b_ref[...])
    oh = oh_ref[0]
    tvec = jnp.dot(oh, te_ref[...], preferred_element_type=f32)
    gl = (jnp.dot(s2, wg1_ref[...], preferred_element_type=f32)
          + jnp.dot(tvec, wg2_ref[...], preferred_element_type=f32)
          + bg_ref[...])
    gl_ref[0] = gl

    eidx = jax.lax.broadcasted_iota(jnp.int32, (NPAD, E), 1)
    m1 = jnp.max(gl, axis=-1, keepdims=True)
    i1 = jnp.min(jnp.where(gl == m1, eidx, E), axis=-1, keepdims=True)
    oh1 = eidx == i1
    gl2 = jnp.where(oh1, -1e30, gl)
    m2 = jnp.max(gl2, axis=-1, keepdims=True)
    i2 = jnp.min(jnp.where(gl2 == m2, eidx, E), axis=-1, keepdims=True)
    sel = oh1 | (eidx == i2)
    es = jnp.where(sel, jnp.exp(gl - m1), 0.0)
    z = jnp.sum(es, axis=-1, keepdims=True)
    gates = es / z
    omega = 1.0 - jnp.max(gates, axis=-1, keepdims=True)

    acc = jnp.zeros((NPAD, D), f32)
    for e in range(E):
        he = _gelu(jnp.dot(s2, we1_ref[e], preferred_element_type=f32)
                   + be1_ref[e])
        eo = jnp.dot(he, we2_ref[e], preferred_element_type=f32) + be2_ref[e]
        acc = acc + gates[:, e:e + 1] * eo

    univ = jnp.dot(_gelu(jnp.dot(s2, wu1_ref[...], preferred_element_type=f32)
                         + bu1_ref[...]),
                   wu2_ref[...], preferred_element_type=f32) + bu2_ref[...]

    src = src + acc + omega * univ
    out = _ln(src, nfg_ref[...], nfb_ref[...])
    cls = out[0:1, :]

    a = jnp.concatenate(
        [jnp.dot(cls, hw_ref[t], preferred_element_type=f32)
         for t in range(T)], axis=0)
    tl = (jnp.dot(oh[:, :T], a, preferred_element_type=f32)
          + jnp.dot(oh[:, :T], hb_ref[...], preferred_element_type=f32))
    tl_ref[0] = tl


def kernel(x, task_ids, params):
    p = params
    B = x.shape[0]
    f32 = jnp.float32

    xs = x.reshape(B, NTOK, SEGLEN)
    row2 = lambda a: a.reshape(1, -1)
    oh = jax.nn.one_hot(task_ids, E, dtype=f32).reshape(B, 1, E)
    te_pad = jnp.pad(p['task_embed'], ((0, E - T), (0, 0)))
    wg1 = p['Wg'][:D]
    wg2 = p['Wg'][D:]

    const = lambda shape: pl.BlockSpec(shape, lambda b: tuple(0 for _ in shape))
    gl_out, tl_out = pl.pallas_call(
        _fused_kernel,
        grid=(B,),
        in_specs=[
            pl.BlockSpec((1, NTOK, SEGLEN), lambda b: (b, 0, 0)),
            pl.BlockSpec((1, 1, E), lambda b: (b, 0, 0)),
            const((SEGLEN, FLAT)), const((SEGLEN, FLAT)),
            const((FLAT, D)), const((1, D)), const((1, D)), const((1, D)),
            const((1, NTOK, D)), const((1, D)),
            const((1, D)), const((1, D)), const((1, D)), const((1, D)),
            const((1, D)), const((1, D)),
            const((D, D)), const((1, D)), const((D, D)), const((1, D)),
            const((D, D)), const((1, D)), const((D, D)), const((1, D)),
            const((E, D)), const((D, E)), const((D, E)), const((1, E)),
            const((E, D, DFF)), const((E, 1, DFF)),
            const((E, DFF, D)), const((E, 1, D)),
            const((D, DFF)), const((1, DFF)), const((DFF, D)), const((1, D)),
            const((T, D, 2)), const((T, 2)),
        ],
        out_specs=[
            pl.BlockSpec((1, NPAD, E), lambda b: (b, 0, 0)),
            pl.BlockSpec((1, 1, 2), lambda b: (b, 0, 0)),
        ],
        out_shape=[
            jax.ShapeDtypeStruct((B, NPAD, E), f32),
            jax.ShapeDtypeStruct((B, 1, 2), f32),
        ],
    )(xs, oh,
      jnp.asarray(_CF), jnp.asarray(_SF), p['W_proj'],
      row2(p['b_proj']), row2(p['ln_proj_g']), row2(p['ln_proj_b']),
      p['pos_embed'], p['cls_token'].reshape(1, D),
      row2(p['norm1_g']), row2(p['norm1_b']),
      row2(p['norm2_g']), row2(p['norm2_b']),
      row2(p['normf_g']), row2(p['normf_b']),
      p['Wq'], row2(p['bq']), p['Wk'], row2(p['bk']),
      p['Wv'], row2(p['bv']), p['Wo'], row2(p['bo']),
      te_pad, wg1, wg2, row2(p['bg']),
      p['We1'], p['be1'].reshape(E, 1, DFF),
      p['We2'], p['be2'].reshape(E, 1, D),
      p['Wu1'], row2(p['bu1']), p['Wu2'], row2(p['bu2']),
      p['head_W'], p['head_b'])

    task_logits = tl_out.reshape(B, 2)
    logits = gl_out[:, :NREAL, :]
    return task_logits, logits


# speedup vs baseline: 3.9663x; 1.0169x over previous
import math

import jax
import jax.numpy as jnp
import numpy as np
from jax.experimental import pallas as pl

D = 128
DFF = 512
E = 8
H = 4
T = 5
NSEG = 30
SEGLEN = 250
NFFT = 64
HOP = 32
NFREQ = 33
NFRAMES = 8
FLAT = NFREQ * NFRAMES
NTOK = 8 * NSEG
NPAD = 256
NREAL = NTOK + 1
PADLEN = SEGLEN + NFFT


def _make_dft_mats():
    x = np.arange(NFFT)
    q = np.arange(NFREQ)[None, :]
    ang = 2.0 * np.pi * x[:, None] * q / NFFT
    C = np.cos(ang)
    S = np.sin(ang)
    CF = np.zeros((SEGLEN, FLAT), np.float64)
    SF = np.zeros((SEGLEN, FLAT), np.float64)
    for fr in range(NFRAMES):
        for xi in range(NFFT):
            jp = fr * HOP + xi
            if jp < NFFT // 2:
                si = NFFT // 2 - jp
            elif jp < NFFT // 2 + SEGLEN:
                si = jp - NFFT // 2
            else:
                si = (SEGLEN - 2) - (jp - (NFFT // 2 + SEGLEN))
            CF[si, q[0] * NFRAMES + fr] += C[xi]
            SF[si, q[0] * NFRAMES + fr] += S[xi]
    return CF.astype(np.float32), SF.astype(np.float32)


_CF, _SF = _make_dft_mats()


def _gelu(x):
    return 0.5 * x * (1.0 + jax.lax.erf(x * (1.0 / math.sqrt(2.0))))


def _ln(x, g, b, eps=1e-5):
    m = jnp.mean(x, axis=-1, keepdims=True)
    v = jnp.mean((x - m) ** 2, axis=-1, keepdims=True)
    return (x - m) * jax.lax.rsqrt(v + eps) * g + b


def _fused_kernel(xs_ref, tid_ref,
                  cb_ref, sb_ref, w2_ref, bp_ref, lng_ref, lnb_ref,
                  pos_ref, cls_ref,
                  n1g_ref, n1b_ref, n2g_ref, n2b_ref, nfg_ref, nfb_ref,
                  wq_ref, bq_ref, wk_ref, bk_ref, wv_ref, bv_ref,
                  wo_ref, bo_ref,
                  te_ref, wg_ref, bg_ref,
                  we1_ref, be1_ref, we2_ref, be2_ref,
                  wu1_ref, bu1_ref, wu2_ref, bu2_ref,
                  hw_ref, hb_ref,
                  gl_ref, tl_ref):
    dh = D // H
    f32 = jnp.float32

    xs = xs_ref[0]
    re = jnp.dot(xs, cb_ref[...], preferred_element_type=f32)
    im = jnp.dot(xs, sb_ref[...], preferred_element_type=f32)
    mag = jnp.sqrt(re * re + im * im)
    tok = jnp.dot(mag, w2_ref[...], preferred_element_type=f32) + bp_ref[...]
    tok = _gelu(_ln(tok, lng_ref[...], lnb_ref[...])) + pos_ref[0]

    src = jnp.concatenate(
        [cls_ref[...], tok, jnp.zeros((NPAD - NREAL, D), f32)], axis=0)

    s2 = _ln(src, n1g_ref[...], n1b_ref[...])
    q = jnp.dot(s2, wq_ref[...], preferred_element_type=f32) + bq_ref[...]
    k = jnp.dot(s2, wk_ref[...], preferred_element_type=f32) + bk_ref[...]
    v = jnp.dot(s2, wv_ref[...], preferred_element_type=f32) + bv_ref[...]

    col = jax.lax.broadcasted_iota(jnp.int32, (NPAD, NPAD), 1)
    key_mask = col < NREAL
    scale = 1.0 / math.sqrt(dh)
    heads = []
    for h in range(H):
        sl = slice(h * dh, (h + 1) * dh)
        sc = jnp.dot(q[:, sl], k[:, sl].T, preferred_element_type=f32) * scale
        sc = jnp.where(key_mask, sc, -1e30)
        m = jnp.max(sc, axis=-1, keepdims=True)
        p = jnp.exp(sc - m)
        p = p / jnp.sum(p, axis=-1, keepdims=True)
        heads.append(jnp.dot(p, v[:, sl], preferred_element_type=f32))
    ao = jnp.dot(jnp.concatenate(heads, axis=1), wo_ref[...],
                 preferred_element_type=f32) + bo_ref[...]
    src = src + ao

    s2 = _ln(src, n2g_ref[...], n2b_ref[...])
    oh = jnp.where(jax.lax.broadcasted_iota(jnp.int32, (1, E), 1) == tid_ref[0],
                   1.0, 0.0)
    tvec = jnp.dot(oh[:, :T], te_ref[...], preferred_element_type=f32)
    gl = (jnp.dot(s2, wg_ref[:D], preferred_element_type=f32)
          + jnp.dot(tvec, wg_ref[D:], preferred_element_type=f32)
          + bg_ref[...])
    gl_ref[0] = gl[:NREAL]

    eidx = jax.lax.broadcasted_iota(jnp.int32, (NPAD, E), 1)
    m1 = jnp.max(gl, axis=-1, keepdims=True)
    i1 = jnp.min(jnp.where(gl == m1, eidx, E), axis=-1, keepdims=True)
    oh1 = eidx == i1
    gl2 = jnp.where(oh1, -1e30, gl)
    m2 = jnp.max(gl2, axis=-1, keepdims=True)
    i2 = jnp.min(jnp.where(gl2 == m2, eidx, E), axis=-1, keepdims=True)
    sel = oh1 | (eidx == i2)
    es = jnp.where(sel, jnp.exp(gl - m1), 0.0)
    z = jnp.sum(es, axis=-1, keepdims=True)
    gates = es / z
    omega = 1.0 - jnp.max(gates, axis=-1, keepdims=True)

    acc = jnp.zeros((NPAD, D), f32)
    for e in range(E):
        he = _gelu(jnp.dot(s2, we1_ref[e], preferred_element_type=f32)
                   + be1_ref[e])
        eo = jnp.dot(he, we2_ref[e], preferred_element_type=f32) + be2_ref[e]
        acc = acc + gates[:, e:e + 1] * eo

    univ = jnp.dot(_gelu(jnp.dot(s2, wu1_ref[...], preferred_element_type=f32)
                         + bu1_ref[...]),
                   wu2_ref[...], preferred_element_type=f32) + bu2_ref[...]

    src = src + acc + omega * univ
    out = _ln(src, nfg_ref[...], nfb_ref[...])
    cls = out[0:1, :]

    a = jnp.concatenate(
        [jnp.dot(cls, hw_ref[t], preferred_element_type=f32)
         for t in range(T)], axis=0)
    tl = (jnp.dot(oh[:, :T], a, preferred_element_type=f32)
          + jnp.dot(oh[:, :T], hb_ref[...], preferred_element_type=f32))
    tl_ref[0] = tl


def kernel(x, task_ids, params):
    p = params
    B = x.shape[0]
    f32 = jnp.float32

    xs = x.reshape(B, NTOK, SEGLEN)
    row2 = lambda a: a.reshape(1, -1)
    tid = task_ids.astype(jnp.int32).reshape(B, 1, 1)

    const = lambda shape: pl.BlockSpec(shape, lambda b: tuple(0 for _ in shape))
    gl_out, tl_out = pl.pallas_call(
        _fused_kernel,
        grid=(B,),
        in_specs=[
            pl.BlockSpec((1, NTOK, SEGLEN), lambda b: (b, 0, 0)),
            pl.BlockSpec((1, 1, 1), lambda b: (b, 0, 0)),
            const((SEGLEN, FLAT)), const((SEGLEN, FLAT)),
            const((FLAT, D)), const((1, D)), const((1, D)), const((1, D)),
            const((1, NTOK, D)), const((1, D)),
            const((1, D)), const((1, D)), const((1, D)), const((1, D)),
            const((1, D)), const((1, D)),
            const((D, D)), const((1, D)), const((D, D)), const((1, D)),
            const((D, D)), const((1, D)), const((D, D)), const((1, D)),
            const((T, D)), const((2 * D, E)), const((1, E)),
            const((E, D, DFF)), const((E, 1, DFF)),
            const((E, DFF, D)), const((E, 1, D)),
            const((D, DFF)), const((1, DFF)), const((DFF, D)), const((1, D)),
            const((T, D, 2)), const((T, 2)),
        ],
        out_specs=[
            pl.BlockSpec((1, NREAL, E), lambda b: (b, 0, 0)),
            pl.BlockSpec((1, 1, 2), lambda b: (b, 0, 0)),
        ],
        out_shape=[
            jax.ShapeDtypeStruct((B, NREAL, E), f32),
            jax.ShapeDtypeStruct((B, 1, 2), f32),
        ],
    )(xs, tid,
      jnp.asarray(_CF), jnp.asarray(_SF), p['W_proj'],
      row2(p['b_proj']), row2(p['ln_proj_g']), row2(p['ln_proj_b']),
      p['pos_embed'], p['cls_token'].reshape(1, D),
      row2(p['norm1_g']), row2(p['norm1_b']),
      row2(p['norm2_g']), row2(p['norm2_b']),
      row2(p['normf_g']), row2(p['normf_b']),
      p['Wq'], row2(p['bq']), p['Wk'], row2(p['bk']),
      p['Wv'], row2(p['bv']), p['Wo'], row2(p['bo']),
      p['task_embed'], p['Wg'], row2(p['bg']),
      p['We1'], p['be1'].reshape(E, 1, DFF),
      p['We2'], p['be2'].reshape(E, 1, D),
      p['Wu1'], row2(p['bu1']), p['Wu2'], row2(p['bu2']),
      p['head_W'], p['head_b'])

    return tl_out.reshape(B, 2), gl_out
